# baseline (speedup 1.0000x reference)
import functools

import jax
import jax.numpy as jnp
from jax import lax
from jax.experimental import pallas as pl
from jax.experimental.pallas import tpu as pltpu
from jax.experimental.pallas import tpu_sc as plsc

N = 10000
E = 320000
D = 128
NC = 2
NS = 16
NW = NC * NS
EPW = E // NW
CH = 80
NCH = EPW // CH
NBLK = 5
CPB = NCH // NBLK
NP = N
RPS = 624
TAIL = N - NS * RPS
ZTAIL = NP - NS * RPS
B = 400
GRID = N // B

_mesh = plsc.VectorSubcoreMesh(core_axis_name="c", subcore_axis_name="s")


@functools.partial(
    pl.kernel,
    mesh=_mesh,
    out_type=jax.ShapeDtypeStruct((NC, N, D), jnp.float32),
    scratch_types=[
        pltpu.VMEM((CPB, CH), jnp.int32),
        pltpu.VMEM((CPB, CH), jnp.int32),
        pltpu.VMEM((CPB, CH), jnp.int32),
        pltpu.VMEM((CPB, CH), jnp.int32),
        pltpu.VMEM((CH, D), jnp.float32),
        pltpu.VMEM((CH, D), jnp.float32),
        pltpu.VMEM((CH, D), jnp.float32),
        pltpu.VMEM_SHARED((NP, D), jnp.float32),
        pltpu.SemaphoreType.DMA,
        pltpu.SemaphoreType.DMA,
        pltpu.SemaphoreType.DMA,
        pltpu.SemaphoreType.DMA,
        pltpu.SemaphoreType.DMA,
        pltpu.SemaphoreType.DMA,
        pltpu.SemaphoreType.DMA,
        pltpu.SemaphoreType.DMA,
    ],
)
def _agg(h_hbm, src_hbm, dst_hbm, zeros_hbm, out_hbm,
         sidx0, sidx1, didx0, didx1, rows0, rows1, rows2, acc,
         sg0, sg1, sg2, ss0, ss1, ss2, sz, si):
    c = lax.axis_index("c")
    s = lax.axis_index("s")
    wid = c * NS + s
    rb = s * RPS

    SI = (sidx0, sidx1)
    DI = (didx0, didx1)
    R = (rows0, rows1, rows2)
    SG = (sg0, sg1, sg2)
    SS = (ss0, ss1, ss2)

    def start_gather(j, buf, sem, sidx):
        pltpu.make_async_copy(h_hbm.at[sidx.at[j]], buf, sem).start()

    def start_scatter(j, buf, sem, didx):
        pltpu.make_async_copy(buf, acc.at[didx.at[j]], sem).start(add=True)
        pltpu.make_async_copy(buf, acc.at[didx.at[j]], sem).start(add=True)

    def drain(buf, sem):
        pltpu.make_async_copy(h_hbm.at[pl.ds(0, CH)], buf, sem).wait()

    def drain_s2(buf, sem):
        pltpu.make_async_copy(h_hbm.at[pl.ds(0, CH)], buf, sem).wait()
        pltpu.make_async_copy(h_hbm.at[pl.ds(0, CH)], buf, sem).wait()

    def drain_idx(buf, sem):
        pltpu.make_async_copy(src_hbm.at[wid, 0], buf, sem).wait()

    pltpu.make_async_copy(zeros_hbm.at[pl.ds(rb, RPS)],
                          acc.at[pl.ds(rb, RPS)], sz).start()

    @pl.when(s == NS - 1)
    def _():
        pltpu.make_async_copy(zeros_hbm.at[pl.ds(NS * RPS, ZTAIL)],
                              acc.at[pl.ds(NS * RPS, ZTAIL)], sz).start()

    pltpu.sync_copy(src_hbm.at[wid, 0], SI[0])
    pltpu.sync_copy(dst_hbm.at[wid, 0], DI[0])
    start_gather(0, R[0], SG[0], SI[0])
    start_gather(1, R[1], SG[1], SI[0])
    pltpu.make_async_copy(zeros_hbm.at[pl.ds(rb, RPS)],
                          acc.at[pl.ds(rb, RPS)], sz).wait()

    @pl.when(s == NS - 1)
    def _():
        pltpu.make_async_copy(zeros_hbm.at[pl.ds(NS * RPS, ZTAIL)],
                              acc.at[pl.ds(NS * RPS, ZTAIL)], sz).wait()

    plsc.subcore_barrier()

    pltpu.make_async_copy(src_hbm.at[wid, 1], SI[1], si).start()
    pltpu.make_async_copy(dst_hbm.at[wid, 1], DI[1], si).start()

    for b in range(NBLK):
        cur, nxt = b % 2, (b + 1) % 2
        last = b == NBLK - 1

        def body(u, carry2, b=b, cur=cur, nxt=nxt, last=last):
            for t in range(3):
                j = 3 * u + t
                bt = (b + t) % 3
                nb = (b + t + 2) % 3
                drain(R[bt], SG[bt])
                start_scatter(j, R[bt], SS[bt], DI[cur])
                if b == 0 and t == 0:
                    @pl.when(u > 0)
                    def _():
                        drain_s2(R[nb], SS[nb])
                else:
                    drain_s2(R[nb], SS[nb])
                if b > 0 and t == 0:
                    if not last:
                        @pl.when(u == 0)
                        def _():
                            pltpu.make_async_copy(
                                src_hbm.at[wid, b + 1], SI[nxt], si).start()
                            pltpu.make_async_copy(
                                dst_hbm.at[wid, b + 1], DI[nxt], si).start()

                if t == 2:
                    @pl.when(u < CPB // 3 - 1)
                    def _(j=j, nb=nb, cur=cur):
                        start_gather(j + 2, R[nb], SG[nb], SI[cur])

                    if not last:
                        @pl.when(u == CPB // 3 - 1)
                        def _(nb=nb, nxt=nxt):
                            drain_idx(SI[nxt], si)
                            drain_idx(DI[nxt], si)
                            start_gather(0, R[nb], SG[nb], SI[nxt])
                else:
                    start_gather(j + 2, R[nb], SG[nb], SI[cur])
            return carry2

        lax.fori_loop(0, CPB // 3, body, 0)
        b0 = b % 3
        n0 = (b + 2) % 3
        drain(R[b0], SG[b0])
        start_scatter(CPB - 1, R[b0], SS[b0], DI[cur])
        drain_s2(R[n0], SS[n0])
        if not last:
            start_gather(1, R[n0], SG[n0], SI[nxt])

    drain_s2(R[(NBLK - 1) % 3], SS[(NBLK - 1) % 3])
    plsc.subcore_barrier()
    pltpu.sync_copy(acc.at[pl.ds(rb, RPS)], out_hbm.at[c, pl.ds(rb, RPS)])

    @pl.when(s == NS - 1)
    def _():
        pltpu.sync_copy(acc.at[pl.ds(NS * RPS, TAIL)],
                        out_hbm.at[c, pl.ds(NS * RPS, TAIL)])


@functools.partial(
    pl.kernel,
    mesh=_mesh,
    out_type=jax.ShapeDtypeStruct((16, NP), jnp.float32),
    scratch_types=[
        pltpu.VMEM((NBLK, CPB, CH), jnp.int32),
        pltpu.VMEM((CH,), jnp.float32),
        pltpu.VMEM_SHARED((NP,), jnp.float32),
        pltpu.SemaphoreType.DMA,
    ],
)
def _cnt(dst_hbm, zeros_hbm, out_hbm, dst_v, ones_v, acc, sem):
    c = lax.axis_index("c")
    s = lax.axis_index("s")
    wid = c * NS + s
    pltpu.sync_copy(dst_hbm.at[wid], dst_v)
    for k in range(CH // 16):
        ones_v[pl.ds(k * 16, 16)] = jnp.ones((16,), jnp.float32)

    @pl.when(s == 0)
    def _():
        pltpu.sync_copy(zeros_hbm, acc)

    plsc.subcore_barrier()

    def body(b, carry):
        def inner(k, carry2):
            pltpu.sync_copy(ones_v, acc.at[dst_v.at[b, k]], add=True)
            return carry2
        lax.fori_loop(0, CPB, inner, carry)
        return carry

    lax.fori_loop(0, NBLK, body, 0)
    plsc.subcore_barrier()

    @pl.when(s == 0)
    def _():
        pltpu.sync_copy(acc, out_hbm.at[8 * c])


def _enc_body(x_ref, w_ref, b_ref, wl_ref, z_ref, h_ref):
    z = jnp.dot(x_ref[...], w_ref[...], preferred_element_type=jnp.float32)
    z = jnp.maximum(z + b_ref[...], 0.0)
    z_ref[...] = z
    h_ref[...] = jnp.dot(z, wl_ref[...], preferred_element_type=jnp.float32)


def _enc(x, w, b, wl):
    return pl.pallas_call(
        _enc_body,
        grid=(GRID,),
        in_specs=[
            pl.BlockSpec((B, 128), lambda i: (i, 0)),
            pl.BlockSpec((128, 128), lambda i: (0, 0)),
            pl.BlockSpec((1, 128), lambda i: (0, 0)),
            pl.BlockSpec((128, 128), lambda i: (0, 0)),
        ],
        out_specs=[
            pl.BlockSpec((B, 128), lambda i: (i, 0)),
            pl.BlockSpec((B, 128), lambda i: (i, 0)),
        ],
        out_shape=[
            jax.ShapeDtypeStruct((N, 128), jnp.float32),
            jax.ShapeDtypeStruct((N, 128), jnp.float32),
        ],
    )(x, w, b, wl)


def _inv_deg(c_ref):
    cnt = c_ref[0] + c_ref[1]
    return 0.5 / jnp.maximum(cnt, 1.0)


def _comb_body(p_ref, c_ref, z_ref, wr_ref, bl_ref, wln_ref, zn_ref, hn_ref):
    aggr = (p_ref[0] + p_ref[1]) * _inv_deg(c_ref)
    t = aggr + bl_ref[...] + jnp.dot(z_ref[...], wr_ref[...], preferred_element_type=jnp.float32)
    zn = jnp.maximum(t, 0.0)
    zn_ref[...] = zn
    hn_ref[...] = jnp.dot(zn, wln_ref[...], preferred_element_type=jnp.float32)


def _comb(p, cp, z, wr, bl, wln):
    return pl.pallas_call(
        _comb_body,
        grid=(GRID,),
        in_specs=[
            pl.BlockSpec((NC, B, 128), lambda i: (0, i, 0)),
            pl.BlockSpec((NC, B, 1), lambda i: (0, i, 0)),
            pl.BlockSpec((B, 128), lambda i: (i, 0)),
            pl.BlockSpec((128, 128), lambda i: (0, 0)),
            pl.BlockSpec((1, 128), lambda i: (0, 0)),
            pl.BlockSpec((128, 128), lambda i: (0, 0)),
        ],
        out_specs=[
            pl.BlockSpec((B, 128), lambda i: (i, 0)),
            pl.BlockSpec((B, 128), lambda i: (i, 0)),
        ],
        out_shape=[
            jax.ShapeDtypeStruct((N, 128), jnp.float32),
            jax.ShapeDtypeStruct((N, 128), jnp.float32),
        ],
    )(p, cp, z, wr, bl, wln)


def _final_body(p_ref, c_ref, z_ref, wr_ref, bl_ref, wd_ref, bd_ref, o_ref):
    aggr = (p_ref[0, :, :64] + p_ref[1, :, :64]) * _inv_deg(c_ref)
    t = aggr + bl_ref[...] + jnp.dot(z_ref[...], wr_ref[...], preferred_element_type=jnp.float32)
    o_ref[...] = jnp.dot(t, wd_ref[...], preferred_element_type=jnp.float32) + bd_ref[...]


def _final(p, cp, z, wr, bl, wd, bd):
    return pl.pallas_call(
        _final_body,
        grid=(GRID,),
        in_specs=[
            pl.BlockSpec((NC, B, 128), lambda i: (0, i, 0)),
            pl.BlockSpec((NC, B, 1), lambda i: (0, i, 0)),
            pl.BlockSpec((B, 128), lambda i: (i, 0)),
            pl.BlockSpec((128, 64), lambda i: (0, 0)),
            pl.BlockSpec((1, 64), lambda i: (0, 0)),
            pl.BlockSpec((64, 4), lambda i: (0, 0)),
            pl.BlockSpec((1, 4), lambda i: (0, 0)),
        ],
        out_specs=pl.BlockSpec((B, 4), lambda i: (i, 0)),
        out_shape=jax.ShapeDtypeStruct((N, 4), jnp.float32),
    )(p, cp, z, wr, bl, wd, bd)


def kernel(x, edge_index, W_enc, b_enc, Wl0, Wr0, bl0, Wl1, Wr1, bl1, Wl2, Wr2, bl2, Wl3, Wr3, bl3, W_dec, b_dec):
    src = edge_index[0].astype(jnp.int32).reshape(NW, NBLK, CPB, CH)
    dst = edge_index[1].astype(jnp.int32).reshape(NW, NBLK, CPB, CH)
    zeros2d = jnp.zeros((NP, D), jnp.float32)
    zeros1d = jnp.zeros((NP,), jnp.float32)
    Wl3p = jnp.concatenate([Wl3, jnp.zeros((128, 64), jnp.float32)], axis=1)

    cp = _cnt(dst, zeros1d)[0::8, :N].reshape(NC, N, 1)
    z0, h0 = _enc(x, W_enc, b_enc.reshape(1, 128), Wl0)
    p0 = _agg(h0, src, dst, zeros2d)
    z1, h1 = _comb(p0, cp, z0, Wr0, bl0.reshape(1, 128), Wl1)
    p1 = _agg(h1, src, dst, zeros2d)
    z2, h2 = _comb(p1, cp, z1, Wr1, bl1.reshape(1, 128), Wl2)
    p2 = _agg(h2, src, dst, zeros2d)
    z3, h3 = _comb(p2, cp, z2, Wr2, bl2.reshape(1, 128), Wl3p)
    p3 = _agg(h3, src, dst, zeros2d)
    return _final(p3, cp, z3, Wr3, bl3.reshape(1, 64), W_dec, b_dec.reshape(1, 4))


# speedup vs baseline: 1.4444x; 1.4444x over previous
import functools

import jax
import jax.numpy as jnp
from jax import lax
from jax.experimental import pallas as pl
from jax.experimental.pallas import tpu as pltpu
from jax.experimental.pallas import tpu_sc as plsc

N = 10000
E = 320000
D = 128
NC = 2
NS = 16
NW = NC * NS
EPW = E // NW
CH = 80
NCH = EPW // CH
NBLK = 5
CPB = NCH // NBLK
NP = N
RPS = 624
TAIL = N - NS * RPS
ZTAIL = NP - NS * RPS
B = 400
GRID = N // B

_mesh = plsc.VectorSubcoreMesh(core_axis_name="c", subcore_axis_name="s")


@functools.partial(
    pl.kernel,
    mesh=_mesh,
    out_type=jax.ShapeDtypeStruct((NC, N, D), jnp.float32),
    scratch_types=[
        pltpu.VMEM((CPB, CH), jnp.int32),
        pltpu.VMEM((CPB, CH), jnp.int32),
        pltpu.VMEM((CPB, CH), jnp.int32),
        pltpu.VMEM((CPB, CH), jnp.int32),
        pltpu.VMEM((CH, D), jnp.float32),
        pltpu.VMEM((CH, D), jnp.float32),
        pltpu.VMEM((CH, D), jnp.float32),
        pltpu.VMEM_SHARED((NP, D), jnp.float32),
        pltpu.SemaphoreType.DMA,
        pltpu.SemaphoreType.DMA,
        pltpu.SemaphoreType.DMA,
        pltpu.SemaphoreType.DMA,
        pltpu.SemaphoreType.DMA,
        pltpu.SemaphoreType.DMA,
        pltpu.SemaphoreType.DMA,
        pltpu.SemaphoreType.DMA,
    ],
)
def _agg(h_hbm, src_hbm, dst_hbm, zeros_hbm, dum_hbm, out_hbm,
         sidx0, sidx1, didx0, didx1, rows0, rows1, rows2, acc,
         sg0, sg1, sg2, ss0, ss1, ss2, sz, si):
    c = lax.axis_index("c")
    s = lax.axis_index("s")
    wid = c * NS + s
    rb = s * RPS

    SI = (sidx0, sidx1)
    DI = (didx0, didx1)
    R = (rows0, rows1, rows2)
    SG = (sg0, sg1, sg2)
    SS = (ss0, ss1, ss2)

    def start_gather(j, buf, sem, sidx):
        pltpu.make_async_copy(h_hbm.at[sidx.at[j]], buf, sem).start()

    def start_scatter(j, buf, sem, didx):
        pltpu.make_async_copy(buf, acc.at[didx.at[j]], sem).start(add=True)

    def drain(buf, sem):
        pltpu.make_async_copy(dum_hbm, buf, sem).wait()

    def drain_s2(buf, sem):
        pltpu.make_async_copy(dum_hbm, buf, sem).wait()

    def drain_idx(buf, sem):
        pltpu.make_async_copy(src_hbm.at[wid, 0], buf, sem).wait()

    pltpu.make_async_copy(zeros_hbm.at[pl.ds(rb, RPS)],
                          acc.at[pl.ds(rb, RPS)], sz).start()

    @pl.when(s == NS - 1)
    def _():
        pltpu.make_async_copy(zeros_hbm.at[pl.ds(NS * RPS, ZTAIL)],
                              acc.at[pl.ds(NS * RPS, ZTAIL)], sz).start()

    pltpu.sync_copy(src_hbm.at[wid, 0], SI[0])
    pltpu.sync_copy(dst_hbm.at[wid, 0], DI[0])
    start_gather(0, R[0], SG[0], SI[0])
    start_gather(1, R[1], SG[1], SI[0])
    pltpu.make_async_copy(zeros_hbm.at[pl.ds(rb, RPS)],
                          acc.at[pl.ds(rb, RPS)], sz).wait()

    @pl.when(s == NS - 1)
    def _():
        pltpu.make_async_copy(zeros_hbm.at[pl.ds(NS * RPS, ZTAIL)],
                              acc.at[pl.ds(NS * RPS, ZTAIL)], sz).wait()

    plsc.subcore_barrier()

    pltpu.make_async_copy(src_hbm.at[wid, 1], SI[1], si).start()
    pltpu.make_async_copy(dst_hbm.at[wid, 1], DI[1], si).start()

    for b in range(NBLK):
        cur, nxt = b % 2, (b + 1) % 2
        last = b == NBLK - 1

        def body(u, carry2, b=b, cur=cur, nxt=nxt, last=last):
            for t in range(3):
                j = 3 * u + t
                bt = (b + t) % 3
                nb = (b + t + 2) % 3
                drain(R[bt], SG[bt])
                start_scatter(j, R[bt], SS[bt], DI[cur])
                if b == 0 and t == 0:
                    @pl.when(u > 0)
                    def _():
                        drain_s2(R[nb], SS[nb])
                else:
                    drain_s2(R[nb], SS[nb])
                if b > 0 and t == 0:
                    if not last:
                        @pl.when(u == 0)
                        def _():
                            pltpu.make_async_copy(
                                src_hbm.at[wid, b + 1], SI[nxt], si).start()
                            pltpu.make_async_copy(
                                dst_hbm.at[wid, b + 1], DI[nxt], si).start()

                if t == 2:
                    @pl.when(u < CPB // 3 - 1)
                    def _(j=j, nb=nb, cur=cur):
                        start_gather(j + 2, R[nb], SG[nb], SI[cur])

                    if not last:
                        @pl.when(u == CPB // 3 - 1)
                        def _(nb=nb, nxt=nxt):
                            drain_idx(SI[nxt], si)
                            drain_idx(DI[nxt], si)
                            start_gather(0, R[nb], SG[nb], SI[nxt])
                else:
                    start_gather(j + 2, R[nb], SG[nb], SI[cur])
            return carry2

        lax.fori_loop(0, CPB // 3, body, 0)
        b0 = b % 3
        n0 = (b + 2) % 3
        drain(R[b0], SG[b0])
        start_scatter(CPB - 1, R[b0], SS[b0], DI[cur])
        drain_s2(R[n0], SS[n0])
        if not last:
            start_gather(1, R[n0], SG[n0], SI[nxt])

    drain_s2(R[(NBLK - 1) % 3], SS[(NBLK - 1) % 3])
    plsc.subcore_barrier()
    pltpu.sync_copy(acc.at[pl.ds(rb, RPS)], out_hbm.at[c, pl.ds(rb, RPS)])

    @pl.when(s == NS - 1)
    def _():
        pltpu.sync_copy(acc.at[pl.ds(NS * RPS, TAIL)],
                        out_hbm.at[c, pl.ds(NS * RPS, TAIL)])


@functools.partial(
    pl.kernel,
    mesh=_mesh,
    out_type=jax.ShapeDtypeStruct((16, NP), jnp.float32),
    scratch_types=[
        pltpu.VMEM((NBLK, CPB, CH), jnp.int32),
        pltpu.VMEM((CH,), jnp.float32),
        pltpu.VMEM_SHARED((NP,), jnp.float32),
        pltpu.SemaphoreType.DMA,
    ],
)
def _cnt(dst_hbm, zeros_hbm, out_hbm, dst_v, ones_v, acc, sem):
    c = lax.axis_index("c")
    s = lax.axis_index("s")
    wid = c * NS + s
    pltpu.sync_copy(dst_hbm.at[wid], dst_v)
    for k in range(CH // 16):
        ones_v[pl.ds(k * 16, 16)] = jnp.ones((16,), jnp.float32)

    @pl.when(s == 0)
    def _():
        pltpu.sync_copy(zeros_hbm, acc)

    plsc.subcore_barrier()

    def body(b, carry):
        def inner(k, carry2):
            pltpu.sync_copy(ones_v, acc.at[dst_v.at[b, k]], add=True)
            return carry2
        lax.fori_loop(0, CPB, inner, carry)
        return carry

    lax.fori_loop(0, NBLK, body, 0)
    plsc.subcore_barrier()

    @pl.when(s == 0)
    def _():
        pltpu.sync_copy(acc, out_hbm.at[8 * c])


def _enc_body(x_ref, w_ref, b_ref, wl_ref, z_ref, h_ref):
    z = jnp.dot(x_ref[...], w_ref[...], preferred_element_type=jnp.float32)
    z = jnp.maximum(z + b_ref[...], 0.0)
    z_ref[...] = z
    h_ref[...] = jnp.dot(z, wl_ref[...], preferred_element_type=jnp.float32)


def _enc(x, w, b, wl):
    return pl.pallas_call(
        _enc_body,
        grid=(GRID,),
        in_specs=[
            pl.BlockSpec((B, 128), lambda i: (i, 0)),
            pl.BlockSpec((128, 128), lambda i: (0, 0)),
            pl.BlockSpec((1, 128), lambda i: (0, 0)),
            pl.BlockSpec((128, 128), lambda i: (0, 0)),
        ],
        out_specs=[
            pl.BlockSpec((B, 128), lambda i: (i, 0)),
            pl.BlockSpec((B, 128), lambda i: (i, 0)),
        ],
        out_shape=[
            jax.ShapeDtypeStruct((N, 128), jnp.float32),
            jax.ShapeDtypeStruct((N, 128), jnp.float32),
        ],
    )(x, w, b, wl)


def _inv_deg(c_ref):
    cnt = c_ref[0] + c_ref[1]
    return 1.0 / jnp.maximum(cnt, 1.0)


def _comb_body(p_ref, c_ref, z_ref, wr_ref, bl_ref, wln_ref, zn_ref, hn_ref):
    aggr = (p_ref[0] + p_ref[1]) * _inv_deg(c_ref)
    t = aggr + bl_ref[...] + jnp.dot(z_ref[...], wr_ref[...], preferred_element_type=jnp.float32)
    zn = jnp.maximum(t, 0.0)
    zn_ref[...] = zn
    hn_ref[...] = jnp.dot(zn, wln_ref[...], preferred_element_type=jnp.float32)


def _comb(p, cp, z, wr, bl, wln):
    return pl.pallas_call(
        _comb_body,
        grid=(GRID,),
        in_specs=[
            pl.BlockSpec((NC, B, 128), lambda i: (0, i, 0)),
            pl.BlockSpec((NC, B, 1), lambda i: (0, i, 0)),
            pl.BlockSpec((B, 128), lambda i: (i, 0)),
            pl.BlockSpec((128, 128), lambda i: (0, 0)),
            pl.BlockSpec((1, 128), lambda i: (0, 0)),
            pl.BlockSpec((128, 128), lambda i: (0, 0)),
        ],
        out_specs=[
            pl.BlockSpec((B, 128), lambda i: (i, 0)),
            pl.BlockSpec((B, 128), lambda i: (i, 0)),
        ],
        out_shape=[
            jax.ShapeDtypeStruct((N, 128), jnp.float32),
            jax.ShapeDtypeStruct((N, 128), jnp.float32),
        ],
    )(p, cp, z, wr, bl, wln)


def _final_body(p_ref, c_ref, z_ref, wr_ref, bl_ref, wd_ref, bd_ref, o_ref):
    aggr = (p_ref[0, :, :64] + p_ref[1, :, :64]) * _inv_deg(c_ref)
    t = aggr + bl_ref[...] + jnp.dot(z_ref[...], wr_ref[...], preferred_element_type=jnp.float32)
    o_ref[...] = jnp.dot(t, wd_ref[...], preferred_element_type=jnp.float32) + bd_ref[...]


def _final(p, cp, z, wr, bl, wd, bd):
    return pl.pallas_call(
        _final_body,
        grid=(GRID,),
        in_specs=[
            pl.BlockSpec((NC, B, 128), lambda i: (0, i, 0)),
            pl.BlockSpec((NC, B, 1), lambda i: (0, i, 0)),
            pl.BlockSpec((B, 128), lambda i: (i, 0)),
            pl.BlockSpec((128, 64), lambda i: (0, 0)),
            pl.BlockSpec((1, 64), lambda i: (0, 0)),
            pl.BlockSpec((64, 4), lambda i: (0, 0)),
            pl.BlockSpec((1, 4), lambda i: (0, 0)),
        ],
        out_specs=pl.BlockSpec((B, 4), lambda i: (i, 0)),
        out_shape=jax.ShapeDtypeStruct((N, 4), jnp.float32),
    )(p, cp, z, wr, bl, wd, bd)


def kernel(x, edge_index, W_enc, b_enc, Wl0, Wr0, bl0, Wl1, Wr1, bl1, Wl2, Wr2, bl2, Wl3, Wr3, bl3, W_dec, b_dec):
    src = edge_index[0].astype(jnp.int32).reshape(NW, NBLK, CPB, CH)
    dst = edge_index[1].astype(jnp.int32).reshape(NW, NBLK, CPB, CH)
    zeros2d = jnp.zeros((NP, D), jnp.float32)
    zeros1d = jnp.zeros((NP,), jnp.float32)
    dum = jnp.zeros((CH, D), jnp.float32)
    Wl3p = jnp.concatenate([Wl3, jnp.zeros((128, 64), jnp.float32)], axis=1)

    cp = _cnt(dst, zeros1d)[0::8, :N].reshape(NC, N, 1)
    z0, h0 = _enc(x, W_enc, b_enc.reshape(1, 128), Wl0)
    p0 = _agg(h0, src, dst, zeros2d, dum)
    z1, h1 = _comb(p0, cp, z0, Wr0, bl0.reshape(1, 128), Wl1)
    p1 = _agg(h1, src, dst, zeros2d, dum)
    z2, h2 = _comb(p1, cp, z1, Wr1, bl1.reshape(1, 128), Wl2)
    p2 = _agg(h2, src, dst, zeros2d, dum)
    z3, h3 = _comb(p2, cp, z2, Wr2, bl2.reshape(1, 128), Wl3p)
    p3 = _agg(h3, src, dst, zeros2d, dum)
    return _final(p3, cp, z3, Wr3, bl3.reshape(1, 64), W_dec, b_dec.reshape(1, 4))
